# baseline (speedup 1.0000x reference)
import functools

import jax
import jax.numpy as jnp
from jax import lax
from jax.experimental import pallas as pl
from jax.experimental.pallas import tpu as pltpu
from jax.experimental.pallas import tpu_sc as plsc

N_NODES = 10000
H = 128
ROW_BLK = 2000


def _leaky(v):
    return jnp.where(v > 0, v, 0.01 * v)


def _pre_body(x_ref, w_ref, b_ref, o_ref):
    o_ref[...] = jnp.dot(x_ref[...], w_ref[...],
                         preferred_element_type=jnp.float32) + b_ref[...]


def _tc_pre(x, wt, b):
    n = x.shape[0]
    grid = n // ROW_BLK
    return pl.pallas_call(
        _pre_body,
        grid=(grid,),
        in_specs=[
            pl.BlockSpec((ROW_BLK, x.shape[1]), lambda i: (i, 0)),
            pl.BlockSpec(wt.shape, lambda i: (0, 0)),
            pl.BlockSpec(b.shape, lambda i: (0, 0)),
        ],
        out_specs=pl.BlockSpec((ROW_BLK, wt.shape[1]), lambda i: (i, 0)),
        out_shape=jax.ShapeDtypeStruct((n, wt.shape[1]), jnp.float32),
    )(x, wt, b)


def _layer_body(h_ref, p0_ref, p1_ref, w1_ref, b1_ref, w2_ref, b2_ref,
                o_ref, st_ref):
    i = pl.program_id(0)
    z = h_ref[...] + p0_ref[...] + p1_ref[...]
    z = _leaky(jnp.dot(z, w1_ref[...], preferred_element_type=jnp.float32)
               + b1_ref[...])
    z = jnp.dot(z, w2_ref[...], preferred_element_type=jnp.float32) + b2_ref[...]
    o_ref[...] = z
    s = jnp.sum(z, axis=0, keepdims=True)
    sq = jnp.sum(z * z, axis=0, keepdims=True)
    part = jnp.concatenate([s, sq, jnp.zeros((6, H), jnp.float32)], axis=0)

    @pl.when(i == 0)
    def _():
        st_ref[...] = part

    @pl.when(i > 0)
    def _():
        st_ref[...] += part


def _tc_layer(h, p0, p1, w1t, b1, w2t, b2):
    n = h.shape[0]
    grid = n // ROW_BLK
    blk = lambda i: (i, 0)
    return pl.pallas_call(
        _layer_body,
        grid=(grid,),
        in_specs=[
            pl.BlockSpec((ROW_BLK, H), blk),
            pl.BlockSpec((ROW_BLK, H), blk),
            pl.BlockSpec((ROW_BLK, H), blk),
            pl.BlockSpec((H, H), lambda i: (0, 0)),
            pl.BlockSpec((1, H), lambda i: (0, 0)),
            pl.BlockSpec((H, H), lambda i: (0, 0)),
            pl.BlockSpec((1, H), lambda i: (0, 0)),
        ],
        out_specs=[
            pl.BlockSpec((ROW_BLK, H), blk),
            pl.BlockSpec((8, H), lambda i: (0, 0)),
        ],
        out_shape=[
            jax.ShapeDtypeStruct((n, H), jnp.float32),
            jax.ShapeDtypeStruct((8, H), jnp.float32),
        ],
        compiler_params=pltpu.CompilerParams(
            dimension_semantics=("arbitrary",)),
    )(h, p0, p1, w1t, b1, w2t, b2)


def _bn_body(h_ref, st_ref, g_ref, b_ref, o_ref):
    s = st_ref[0, :]
    sq = st_ref[1, :]
    mean = s / N_NODES
    var = sq / N_NODES - mean * mean
    scale = g_ref[0, :] * lax.rsqrt(var + 1e-5)
    shift = b_ref[0, :] - mean * scale
    o_ref[...] = h_ref[...] * scale[None, :] + shift[None, :]


def _tc_bn(h, st, g, b):
    n = h.shape[0]
    grid = n // ROW_BLK
    return pl.pallas_call(
        _bn_body,
        grid=(grid,),
        in_specs=[
            pl.BlockSpec((ROW_BLK, H), lambda i: (i, 0)),
            pl.BlockSpec((8, H), lambda i: (0, 0)),
            pl.BlockSpec((1, H), lambda i: (0, 0)),
            pl.BlockSpec((1, H), lambda i: (0, 0)),
        ],
        out_specs=pl.BlockSpec((ROW_BLK, H), lambda i: (i, 0)),
        out_shape=jax.ShapeDtypeStruct((n, H), jnp.float32),
    )(h, st, g, b)


def _post_body(h_ref, w1_ref, b1_ref, w2_ref, b2_ref, o_ref):
    z = _leaky(jnp.dot(h_ref[...], w1_ref[...],
                       preferred_element_type=jnp.float32) + b1_ref[...])
    o_ref[...] = jnp.dot(z, w2_ref[...],
                         preferred_element_type=jnp.float32) + b2_ref[...]


def _tc_post(h, w1t, b1, w2t, b2):
    n = h.shape[0]
    emb = w2t.shape[1]
    grid = n // ROW_BLK
    return pl.pallas_call(
        _post_body,
        grid=(grid,),
        in_specs=[
            pl.BlockSpec((ROW_BLK, H), lambda i: (i, 0)),
            pl.BlockSpec((H, H), lambda i: (0, 0)),
            pl.BlockSpec((1, H), lambda i: (0, 0)),
            pl.BlockSpec((H, emb), lambda i: (0, 0)),
            pl.BlockSpec((1, emb), lambda i: (0, 0)),
        ],
        out_specs=pl.BlockSpec((ROW_BLK, emb), lambda i: (i, 0)),
        out_shape=jax.ShapeDtypeStruct((n, emb), jnp.float32),
    )(h, w1t, b1, w2t, b2)


NC, NS = 2, 16
NW = NC * NS
CH = 80
NCH = 125
AGGN = 10240
ROWS_PER_TILE = AGGN // NS
ZCH = 80


def _sc_body(h_hbm, packed_hbm, out_hbm,
             agg_sh, pidx, u0, u1, rows0, rows1,
             semg0, semg1, sems0, sems1):
    c = lax.axis_index("c")
    s = lax.axis_index("s")
    wid = c * NS + s
    zeros16 = jnp.zeros((16,), jnp.float32)
    mask14 = jnp.full((16,), 16383, jnp.int32)
    sh14 = jnp.full((16,), 14, jnp.int32)

    def _zrow(r, _):
        for cb in range(H // 16):
            rows0[r, pl.ds(cb * 16, 16)] = zeros16
            rows1[r, pl.ds(cb * 16, 16)] = zeros16
        return 0

    lax.fori_loop(0, ZCH, _zrow, 0)
    r0 = s * ROWS_PER_TILE
    nz = ROWS_PER_TILE // ZCH
    for k in range(nz):
        buf = rows0 if k % 2 == 0 else rows1
        pltpu.sync_copy(buf.at[pl.ds(0, ZCH), :],
                        agg_sh.at[pl.ds(r0 + k * ZCH, ZCH), :])
    plsc.subcore_barrier()

    ub = list(range(0, CH, 16))

    def _unpack(j, u):
        for o in ub:
            v = pidx[j, pl.ds(o, 16)]
            u[0, pl.ds(o, 16)] = jnp.bitwise_and(v, mask14)
            u[1, pl.ds(o, 16)] = lax.shift_right_logical(v, sh14)

    def _gather(u, rows, sem):
        pltpu.async_copy(h_hbm.at[u.at[0]], rows, sem)

    def _gwait(u, rows, sem):
        pltpu.make_async_copy(h_hbm.at[u.at[0]], rows, sem).wait()

    def _scat(u, rows, sem):
        pltpu.async_copy(rows, agg_sh.at[u.at[1]], sem, add=True)

    def _swait(u, rows, sem):
        pltpu.make_async_copy(rows, agg_sh.at[u.at[1]], sem).wait()

    pltpu.sync_copy(packed_hbm.at[wid], pidx)

    _unpack(0, u0)
    _gather(u0, rows0, semg0)
    _unpack(1, u1)
    _gather(u1, rows1, semg1)

    def _pair(jj, _):
        j0 = jj * 2
        _gwait(u0, rows0, semg0)
        _scat(u0, rows0, sems0)
        _gwait(u1, rows1, semg1)
        _scat(u1, rows1, sems1)
        _swait(u0, rows0, sems0)
        _unpack(j0 + 2, u0)
        _gather(u0, rows0, semg0)
        _swait(u1, rows1, sems1)
        _unpack(j0 + 3, u1)
        _gather(u1, rows1, semg1)
        return 0

    lax.fori_loop(0, (NCH - 3) // 2, _pair, 0)
    _gwait(u0, rows0, semg0)
    _scat(u0, rows0, sems0)
    _gwait(u1, rows1, semg1)
    _scat(u1, rows1, sems1)
    _swait(u0, rows0, sems0)
    _unpack(NCH - 1, u0)
    _gather(u0, rows0, semg0)
    _swait(u1, rows1, sems1)
    _gwait(u0, rows0, semg0)
    _scat(u0, rows0, sems0)
    _swait(u0, rows0, sems0)
    plsc.subcore_barrier()

    nd = ROWS_PER_TILE // ZCH
    descs = {}
    for k in range(nd):
        rr = r0 + k * ZCH
        buf = rows0 if k % 2 == 0 else rows1
        sem = semg0 if k % 2 == 0 else semg1
        if k >= 2:
            descs[k - 2].wait()
        pltpu.sync_copy(agg_sh.at[pl.ds(rr, ZCH), :], buf.at[pl.ds(0, ZCH), :])
        descs[k] = pltpu.async_copy(buf.at[pl.ds(0, ZCH), :],
                                    out_hbm.at[c, pl.ds(rr, ZCH), :], sem)
    descs[nd - 2].wait()
    descs[nd - 1].wait()


@functools.partial(
    pl.kernel,
    out_type=jax.ShapeDtypeStruct((NC, AGGN, H), jnp.float32),
    mesh=plsc.VectorSubcoreMesh(core_axis_name="c", subcore_axis_name="s"),
    scratch_types=[
        pltpu.VMEM_SHARED((AGGN, H), jnp.float32),
        pltpu.VMEM((NCH, CH), jnp.int32),
        pltpu.VMEM((2, CH), jnp.int32),
        pltpu.VMEM((2, CH), jnp.int32),
        pltpu.VMEM((CH, H), jnp.float32),
        pltpu.VMEM((CH, H), jnp.float32),
        pltpu.SemaphoreType.DMA,
        pltpu.SemaphoreType.DMA,
        pltpu.SemaphoreType.DMA,
        pltpu.SemaphoreType.DMA,
    ],
)
def _sc_segsum_call(h_hbm, packed_hbm, out_hbm,
                    agg_sh, pidx, u0, u1, rows0, rows1,
                    semg0, semg1, sems0, sems1):
    _sc_body(h_hbm, packed_hbm, out_hbm,
             agg_sh, pidx, u0, u1, rows0, rows1,
             semg0, semg1, sems0, sems1)


def _segsum(h, packed3):
    out = _sc_segsum_call(h, packed3)
    return out[0], out[1]


def kernel(x, edge_index, batch, pre_W, pre_b, convW1, convb1, convW2,
           convb2, bn_g, bn_b, postW1, postb1, postW2, postb2):
    num_genes, emb = 1000, 64
    L = convW1.shape[0]
    src3 = edge_index[0].reshape(NW, NCH, CH)
    dst3 = edge_index[1].reshape(NW, NCH, CH)
    packed3 = jnp.bitwise_or(src3, jnp.left_shift(dst3, 14))

    h = _tc_pre(x, pre_W.T, pre_b.reshape(1, -1))
    for i in range(L):
        p0, p1 = _segsum(h, packed3)
        h_raw, st = _tc_layer(h, p0, p1, convW1[i].T,
                              convb1[i].reshape(1, -1), convW2[i].T,
                              convb2[i].reshape(1, -1))
        if i < L - 1:
            h = _tc_bn(h_raw, st, bn_g[i].reshape(1, -1),
                       bn_b[i].reshape(1, -1))
        else:
            h = h_raw
    out = _tc_post(h, postW1.T, postb1.reshape(1, -1), postW2.T,
                   postb2.reshape(1, -1))
    return out.reshape(-1, num_genes * emb)


# speedup vs baseline: 1.3104x; 1.3104x over previous
import functools

import jax
import jax.numpy as jnp
from jax import lax
from jax.experimental import pallas as pl
from jax.experimental.pallas import tpu as pltpu
from jax.experimental.pallas import tpu_sc as plsc

N_NODES = 10000
H = 128
ROW_BLK = 2000


def _leaky(v):
    return jnp.where(v > 0, v, 0.01 * v)


def _pre_body(x_ref, w_ref, b_ref, o_ref):
    o_ref[...] = jnp.dot(x_ref[...], w_ref[...],
                         preferred_element_type=jnp.float32) + b_ref[...]


def _tc_pre(x, wt, b):
    n = x.shape[0]
    grid = n // ROW_BLK
    return pl.pallas_call(
        _pre_body,
        grid=(grid,),
        in_specs=[
            pl.BlockSpec((ROW_BLK, x.shape[1]), lambda i: (i, 0)),
            pl.BlockSpec(wt.shape, lambda i: (0, 0)),
            pl.BlockSpec(b.shape, lambda i: (0, 0)),
        ],
        out_specs=pl.BlockSpec((ROW_BLK, wt.shape[1]), lambda i: (i, 0)),
        out_shape=jax.ShapeDtypeStruct((n, wt.shape[1]), jnp.float32),
    )(x, wt, b)


def _layer_body(h_ref, p_ref, w1_ref, b1_ref, w2_ref, b2_ref,
                o_ref, st_ref):
    i = pl.program_id(0)
    z = h_ref[...] + p_ref[0] + p_ref[1]
    z = _leaky(jnp.dot(z, w1_ref[...], preferred_element_type=jnp.float32)
               + b1_ref[...])
    z = jnp.dot(z, w2_ref[...], preferred_element_type=jnp.float32) + b2_ref[...]
    o_ref[...] = z
    s = jnp.sum(z, axis=0, keepdims=True)
    sq = jnp.sum(z * z, axis=0, keepdims=True)
    part = jnp.concatenate([s, sq, jnp.zeros((6, H), jnp.float32)], axis=0)

    @pl.when(i == 0)
    def _():
        st_ref[...] = part

    @pl.when(i > 0)
    def _():
        st_ref[...] += part


def _tc_layer(h, parts, w1t, b1, w2t, b2):
    n = h.shape[0]
    grid = n // ROW_BLK
    return pl.pallas_call(
        _layer_body,
        grid=(grid,),
        in_specs=[
            pl.BlockSpec((ROW_BLK, H), lambda i: (i, 0)),
            pl.BlockSpec((2, ROW_BLK, H), lambda i: (0, i, 0)),
            pl.BlockSpec((H, H), lambda i: (0, 0)),
            pl.BlockSpec((1, H), lambda i: (0, 0)),
            pl.BlockSpec((H, H), lambda i: (0, 0)),
            pl.BlockSpec((1, H), lambda i: (0, 0)),
        ],
        out_specs=[
            pl.BlockSpec((ROW_BLK, H), lambda i: (i, 0)),
            pl.BlockSpec((8, H), lambda i: (0, 0)),
        ],
        out_shape=[
            jax.ShapeDtypeStruct((n, H), jnp.float32),
            jax.ShapeDtypeStruct((8, H), jnp.float32),
        ],
        compiler_params=pltpu.CompilerParams(
            dimension_semantics=("arbitrary",)),
    )(h, parts, w1t, b1, w2t, b2)


def _layer_post_body(h_ref, p_ref, w1_ref, b1_ref, w2_ref, b2_ref,
                     pw1_ref, pb1_ref, pw2_ref, pb2_ref, o_ref):
    z = h_ref[...] + p_ref[0] + p_ref[1]
    z = _leaky(jnp.dot(z, w1_ref[...], preferred_element_type=jnp.float32)
               + b1_ref[...])
    z = jnp.dot(z, w2_ref[...], preferred_element_type=jnp.float32) + b2_ref[...]
    z = _leaky(jnp.dot(z, pw1_ref[...], preferred_element_type=jnp.float32)
               + pb1_ref[...])
    o_ref[...] = jnp.dot(z, pw2_ref[...],
                         preferred_element_type=jnp.float32) + pb2_ref[...]


def _tc_layer_post(h, parts, w1t, b1, w2t, b2, pw1t, pb1, pw2t, pb2):
    n = h.shape[0]
    emb = pw2t.shape[1]
    grid = n // ROW_BLK
    full = lambda i: (0, 0)
    return pl.pallas_call(
        _layer_post_body,
        grid=(grid,),
        in_specs=[
            pl.BlockSpec((ROW_BLK, H), lambda i: (i, 0)),
            pl.BlockSpec((2, ROW_BLK, H), lambda i: (0, i, 0)),
            pl.BlockSpec((H, H), full),
            pl.BlockSpec((1, H), full),
            pl.BlockSpec((H, H), full),
            pl.BlockSpec((1, H), full),
            pl.BlockSpec((H, H), full),
            pl.BlockSpec((1, H), full),
            pl.BlockSpec((H, emb), full),
            pl.BlockSpec((1, emb), full),
        ],
        out_specs=pl.BlockSpec((ROW_BLK, emb), lambda i: (i, 0)),
        out_shape=jax.ShapeDtypeStruct((n, emb), jnp.float32),
    )(h, parts, w1t, b1, w2t, b2, pw1t, pb1, pw2t, pb2)


def _bn_body(h_ref, st_ref, g_ref, b_ref, o_ref):
    s = st_ref[0, :]
    sq = st_ref[1, :]
    mean = s / N_NODES
    var = sq / N_NODES - mean * mean
    scale = g_ref[0, :] * lax.rsqrt(var + 1e-5)
    shift = b_ref[0, :] - mean * scale
    o_ref[...] = h_ref[...] * scale[None, :] + shift[None, :]


def _tc_bn(h, st, g, b):
    n = h.shape[0]
    grid = n // ROW_BLK
    return pl.pallas_call(
        _bn_body,
        grid=(grid,),
        in_specs=[
            pl.BlockSpec((ROW_BLK, H), lambda i: (i, 0)),
            pl.BlockSpec((8, H), lambda i: (0, 0)),
            pl.BlockSpec((1, H), lambda i: (0, 0)),
            pl.BlockSpec((1, H), lambda i: (0, 0)),
        ],
        out_specs=pl.BlockSpec((ROW_BLK, H), lambda i: (i, 0)),
        out_shape=jax.ShapeDtypeStruct((n, H), jnp.float32),
    )(h, st, g, b)


def _post_body(h_ref, w1_ref, b1_ref, w2_ref, b2_ref, o_ref):
    z = _leaky(jnp.dot(h_ref[...], w1_ref[...],
                       preferred_element_type=jnp.float32) + b1_ref[...])
    o_ref[...] = jnp.dot(z, w2_ref[...],
                         preferred_element_type=jnp.float32) + b2_ref[...]


def _tc_post(h, w1t, b1, w2t, b2):
    n = h.shape[0]
    emb = w2t.shape[1]
    grid = n // ROW_BLK
    return pl.pallas_call(
        _post_body,
        grid=(grid,),
        in_specs=[
            pl.BlockSpec((ROW_BLK, H), lambda i: (i, 0)),
            pl.BlockSpec((H, H), lambda i: (0, 0)),
            pl.BlockSpec((1, H), lambda i: (0, 0)),
            pl.BlockSpec((H, emb), lambda i: (0, 0)),
            pl.BlockSpec((1, emb), lambda i: (0, 0)),
        ],
        out_specs=pl.BlockSpec((ROW_BLK, emb), lambda i: (i, 0)),
        out_shape=jax.ShapeDtypeStruct((n, emb), jnp.float32),
    )(h, w1t, b1, w2t, b2)


NC, NS = 2, 16
NW = NC * NS
CH = 80
NCH = 125
AGGN = 10240
ROWS_PER_TILE = AGGN // NS
ZCH = 80


def _sc_body(h_hbm, packed_hbm, out_hbm,
             agg_sh, pidx, u0, u1, rows0, rows1,
             semg0, semg1, sems0, sems1):
    c = lax.axis_index("c")
    s = lax.axis_index("s")
    wid = c * NS + s
    zeros16 = jnp.zeros((16,), jnp.float32)
    mask14 = jnp.full((16,), 16383, jnp.int32)
    sh14 = jnp.full((16,), 14, jnp.int32)

    def _zrow(r, _):
        for cb in range(H // 16):
            rows0[r, pl.ds(cb * 16, 16)] = zeros16
            rows1[r, pl.ds(cb * 16, 16)] = zeros16
        return 0

    lax.fori_loop(0, ZCH, _zrow, 0)
    r0 = s * ROWS_PER_TILE
    nz = ROWS_PER_TILE // ZCH
    for k in range(nz):
        buf = rows0 if k % 2 == 0 else rows1
        pltpu.sync_copy(buf.at[pl.ds(0, ZCH), :],
                        agg_sh.at[pl.ds(r0 + k * ZCH, ZCH), :])
    plsc.subcore_barrier()

    ub = list(range(0, CH, 16))

    def _unpack(j, u):
        for o in ub:
            v = pidx[j, pl.ds(o, 16)]
            u[0, pl.ds(o, 16)] = jnp.bitwise_and(v, mask14)
            u[1, pl.ds(o, 16)] = lax.shift_right_logical(v, sh14)

    def _gather(u, rows, sem):
        pltpu.async_copy(h_hbm.at[u.at[0]], rows, sem)

    def _gwait(u, rows, sem):
        pltpu.make_async_copy(h_hbm.at[u.at[0]], rows, sem).wait()

    def _scatter(u, rows):
        pltpu.sync_copy(rows, agg_sh.at[u.at[1]], add=True)

    pltpu.sync_copy(packed_hbm.at[wid], pidx)

    _unpack(0, u0)
    _gather(u0, rows0, semg0)
    _unpack(1, u1)

    def _pair(jj, _):
        j0 = jj * 2
        _gather(u1, rows1, semg1)
        _gwait(u0, rows0, semg0)
        _scatter(u0, rows0)
        _unpack(j0 + 2, u0)
        _gather(u0, rows0, semg0)
        _gwait(u1, rows1, semg1)
        _scatter(u1, rows1)
        _unpack(j0 + 3, u1)
        return 0

    lax.fori_loop(0, (NCH - 3) // 2, _pair, 0)
    _gather(u1, rows1, semg1)
    _gwait(u0, rows0, semg0)
    _scatter(u0, rows0)
    _unpack(NCH - 1, u0)
    _gather(u0, rows0, semg0)
    _gwait(u1, rows1, semg1)
    _scatter(u1, rows1)
    _gwait(u0, rows0, semg0)
    _scatter(u0, rows0)
    plsc.subcore_barrier()

    nd = ROWS_PER_TILE // ZCH
    descs = {}
    for k in range(nd):
        rr = r0 + k * ZCH
        buf = rows0 if k % 2 == 0 else rows1
        sem = semg0 if k % 2 == 0 else semg1
        if k >= 2:
            descs[k - 2].wait()
        pltpu.sync_copy(agg_sh.at[pl.ds(rr, ZCH), :], buf.at[pl.ds(0, ZCH), :])
        descs[k] = pltpu.async_copy(buf.at[pl.ds(0, ZCH), :],
                                    out_hbm.at[c, pl.ds(rr, ZCH), :], sem)
    descs[nd - 2].wait()
    descs[nd - 1].wait()


@functools.partial(
    pl.kernel,
    out_type=jax.ShapeDtypeStruct((NC, AGGN, H), jnp.float32),
    mesh=plsc.VectorSubcoreMesh(core_axis_name="c", subcore_axis_name="s"),
    scratch_types=[
        pltpu.VMEM_SHARED((AGGN, H), jnp.float32),
        pltpu.VMEM((NCH, CH), jnp.int32),
        pltpu.VMEM((2, CH), jnp.int32),
        pltpu.VMEM((2, CH), jnp.int32),
        pltpu.VMEM((CH, H), jnp.float32),
        pltpu.VMEM((CH, H), jnp.float32),
        pltpu.SemaphoreType.DMA,
        pltpu.SemaphoreType.DMA,
        pltpu.SemaphoreType.DMA,
        pltpu.SemaphoreType.DMA,
    ],
)
def _sc_segsum_call(h_hbm, packed_hbm, out_hbm,
                    agg_sh, pidx, u0, u1, rows0, rows1,
                    semg0, semg1, sems0, sems1):
    _sc_body(h_hbm, packed_hbm, out_hbm,
             agg_sh, pidx, u0, u1, rows0, rows1,
             semg0, semg1, sems0, sems1)


def _segsum(h, packed3):
    return _sc_segsum_call(h, packed3)


def kernel(x, edge_index, batch, pre_W, pre_b, convW1, convb1, convW2,
           convb2, bn_g, bn_b, postW1, postb1, postW2, postb2):
    num_genes, emb = 1000, 64
    L = convW1.shape[0]
    src3 = edge_index[0].reshape(NW, NCH, CH)
    dst3 = edge_index[1].reshape(NW, NCH, CH)
    packed3 = jnp.bitwise_or(src3, jnp.left_shift(dst3, 14))

    h = _tc_pre(x, pre_W.T, pre_b.reshape(1, -1))
    for i in range(L - 1):
        parts = _segsum(h, packed3)
        h_raw, st = _tc_layer(h, parts, convW1[i].T,
                              convb1[i].reshape(1, -1), convW2[i].T,
                              convb2[i].reshape(1, -1))
        h = _tc_bn(h_raw, st, bn_g[i].reshape(1, -1),
                   bn_b[i].reshape(1, -1))
    parts = _segsum(h, packed3)
    out = _tc_layer_post(h, parts, convW1[L - 1].T,
                         convb1[L - 1].reshape(1, -1), convW2[L - 1].T,
                         convb2[L - 1].reshape(1, -1),
                         postW1.T, postb1.reshape(1, -1), postW2.T,
                         postb2.reshape(1, -1))
    return out.reshape(-1, num_genes * emb)


# speedup vs baseline: 1.3114x; 1.0007x over previous
import functools

import jax
import jax.numpy as jnp
from jax import lax
from jax.experimental import pallas as pl
from jax.experimental.pallas import tpu as pltpu
from jax.experimental.pallas import tpu_sc as plsc

N_NODES = 10000
H = 128
ROW_BLK = 2000


def _leaky(v):
    return jnp.where(v > 0, v, 0.01 * v)


def _pre_body(x_ref, w_ref, b_ref, o_ref):
    o_ref[...] = jnp.dot(x_ref[...], w_ref[...],
                         preferred_element_type=jnp.float32) + b_ref[...]


def _tc_pre(x, wt, b):
    n = x.shape[0]
    grid = n // ROW_BLK
    return pl.pallas_call(
        _pre_body,
        grid=(grid,),
        in_specs=[
            pl.BlockSpec((ROW_BLK, x.shape[1]), lambda i: (i, 0)),
            pl.BlockSpec(wt.shape, lambda i: (0, 0)),
            pl.BlockSpec(b.shape, lambda i: (0, 0)),
        ],
        out_specs=pl.BlockSpec((ROW_BLK, wt.shape[1]), lambda i: (i, 0)),
        out_shape=jax.ShapeDtypeStruct((n, wt.shape[1]), jnp.float32),
    )(x, wt, b)


def _layer_body(h_ref, p_ref, w1_ref, b1_ref, w2_ref, b2_ref,
                o_ref, st_ref):
    i = pl.program_id(0)
    z = h_ref[...] + p_ref[0] + p_ref[1]
    z = _leaky(jnp.dot(z, w1_ref[...], preferred_element_type=jnp.float32)
               + b1_ref[...])
    z = jnp.dot(z, w2_ref[...], preferred_element_type=jnp.float32) + b2_ref[...]
    o_ref[...] = z
    s = jnp.sum(z, axis=0, keepdims=True)
    sq = jnp.sum(z * z, axis=0, keepdims=True)
    part = jnp.concatenate([s, sq, jnp.zeros((6, H), jnp.float32)], axis=0)

    @pl.when(i == 0)
    def _():
        st_ref[...] = part

    @pl.when(i > 0)
    def _():
        st_ref[...] += part


def _tc_layer(h, parts, w1t, b1, w2t, b2):
    n = h.shape[0]
    grid = n // ROW_BLK
    full = lambda i: (0, 0)
    return pl.pallas_call(
        _layer_body,
        grid=(grid,),
        in_specs=[
            pl.BlockSpec((ROW_BLK, H), lambda i: (i, 0)),
            pl.BlockSpec((2, ROW_BLK, H), lambda i: (0, i, 0)),
            pl.BlockSpec((H, H), full),
            pl.BlockSpec((1, H), full),
            pl.BlockSpec((H, H), full),
            pl.BlockSpec((1, H), full),
        ],
        out_specs=[
            pl.BlockSpec((ROW_BLK, H), lambda i: (i, 0)),
            pl.BlockSpec((8, H), lambda i: (0, 0)),
        ],
        out_shape=[
            jax.ShapeDtypeStruct((n, H), jnp.float32),
            jax.ShapeDtypeStruct((8, H), jnp.float32),
        ],
        compiler_params=pltpu.CompilerParams(
            dimension_semantics=("arbitrary",)),
    )(h, parts, w1t, b1, w2t, b2)


def _bn_body(h_ref, st_ref, g_ref, b_ref, o_ref):
    su = st_ref[0, :]
    sq = st_ref[1, :]
    mean = su / N_NODES
    var = sq / N_NODES - mean * mean
    scale = g_ref[0, :] * lax.rsqrt(var + 1e-5)
    shift = b_ref[0, :] - mean * scale
    o_ref[...] = h_ref[...] * scale[None, :] + shift[None, :]


def _tc_bn(h, st, g, b):
    n = h.shape[0]
    grid = n // ROW_BLK
    return pl.pallas_call(
        _bn_body,
        grid=(grid,),
        in_specs=[
            pl.BlockSpec((ROW_BLK, H), lambda i: (i, 0)),
            pl.BlockSpec((8, H), lambda i: (0, 0)),
            pl.BlockSpec((1, H), lambda i: (0, 0)),
            pl.BlockSpec((1, H), lambda i: (0, 0)),
        ],
        out_specs=pl.BlockSpec((ROW_BLK, H), lambda i: (i, 0)),
        out_shape=jax.ShapeDtypeStruct((n, H), jnp.float32),
    )(h, st, g, b)


def _layer_post_body(h_ref, p_ref, w1_ref, b1_ref, w2_ref, b2_ref,
                     pw1_ref, pb1_ref, pw2_ref, pb2_ref, o_ref):
    z = h_ref[...] + p_ref[0] + p_ref[1]
    z = _leaky(jnp.dot(z, w1_ref[...], preferred_element_type=jnp.float32)
               + b1_ref[...])
    z = jnp.dot(z, w2_ref[...], preferred_element_type=jnp.float32) + b2_ref[...]
    z = _leaky(jnp.dot(z, pw1_ref[...], preferred_element_type=jnp.float32)
               + pb1_ref[...])
    o_ref[...] = jnp.dot(z, pw2_ref[...],
                         preferred_element_type=jnp.float32) + pb2_ref[...]


def _tc_layer_post(h, parts, w1t, b1, w2t, b2, pw1t, pb1, pw2t, pb2):
    n = h.shape[0]
    emb = pw2t.shape[1]
    grid = n // ROW_BLK
    full = lambda i: (0, 0)
    return pl.pallas_call(
        _layer_post_body,
        grid=(grid,),
        in_specs=[
            pl.BlockSpec((ROW_BLK, H), lambda i: (i, 0)),
            pl.BlockSpec((2, ROW_BLK, H), lambda i: (0, i, 0)),
            pl.BlockSpec((H, H), full),
            pl.BlockSpec((1, H), full),
            pl.BlockSpec((H, H), full),
            pl.BlockSpec((1, H), full),
            pl.BlockSpec((H, H), full),
            pl.BlockSpec((1, H), full),
            pl.BlockSpec((H, emb), full),
            pl.BlockSpec((1, emb), full),
        ],
        out_specs=pl.BlockSpec((ROW_BLK, emb), lambda i: (i, 0)),
        out_shape=jax.ShapeDtypeStruct((n, emb), jnp.float32),
    )(h, parts, w1t, b1, w2t, b2, pw1t, pb1, pw2t, pb2)


NC, NS = 2, 16
NW = NC * NS
CH = 80
NCH = 125
AGGN = 10240
ROWS_PER_TILE = AGGN // NS
ZCH = 80


def _sc_body(h_hbm, packed_hbm, out_hbm,
             agg_sh, pidx, u0, u1, rows0, rows1, semg0, semg1):
    c = lax.axis_index("c")
    s = lax.axis_index("s")
    wid = c * NS + s
    zeros16 = jnp.zeros((16,), jnp.float32)
    mask14 = jnp.full((16,), 16383, jnp.int32)
    sh14 = jnp.full((16,), 14, jnp.int32)

    def _zrow(r, _):
        for cb in range(H // 16):
            rows0[r, pl.ds(cb * 16, 16)] = zeros16
            rows1[r, pl.ds(cb * 16, 16)] = zeros16
        return 0

    lax.fori_loop(0, ZCH, _zrow, 0)
    r0 = s * ROWS_PER_TILE
    nz = ROWS_PER_TILE // ZCH
    for k in range(nz):
        buf = rows0 if k % 2 == 0 else rows1
        pltpu.sync_copy(buf.at[pl.ds(0, ZCH), :],
                        agg_sh.at[pl.ds(r0 + k * ZCH, ZCH), :])
    plsc.subcore_barrier()

    ub = list(range(0, CH, 16))

    def _unpack(j, u):
        for o in ub:
            v = pidx[j, pl.ds(o, 16)]
            u[0, pl.ds(o, 16)] = jnp.bitwise_and(v, mask14)
            u[1, pl.ds(o, 16)] = lax.shift_right_logical(v, sh14)

    def _gather(u, rows, sem):
        pltpu.async_copy(h_hbm.at[u.at[0]], rows, sem)

    def _gwait(u, rows, sem):
        pltpu.make_async_copy(h_hbm.at[u.at[0]], rows, sem).wait()

    def _scatter(u, rows):
        pltpu.sync_copy(rows, agg_sh.at[u.at[1]], add=True)

    pltpu.sync_copy(packed_hbm.at[wid], pidx)

    _unpack(0, u0)
    _gather(u0, rows0, semg0)
    _unpack(1, u1)

    def _pair(jj, _):
        j0 = jj * 2
        _gather(u1, rows1, semg1)
        _gwait(u0, rows0, semg0)
        _scatter(u0, rows0)
        _unpack(j0 + 2, u0)
        _gather(u0, rows0, semg0)
        _gwait(u1, rows1, semg1)
        _scatter(u1, rows1)
        _unpack(j0 + 3, u1)
        return 0

    lax.fori_loop(0, (NCH - 3) // 2, _pair, 0)
    _gather(u1, rows1, semg1)
    _gwait(u0, rows0, semg0)
    _scatter(u0, rows0)
    _unpack(NCH - 1, u0)
    _gather(u0, rows0, semg0)
    _gwait(u1, rows1, semg1)
    _scatter(u1, rows1)
    _gwait(u0, rows0, semg0)
    _scatter(u0, rows0)
    plsc.subcore_barrier()

    nd = ROWS_PER_TILE // ZCH
    descs = {}
    for k in range(nd):
        rr = r0 + k * ZCH
        buf = rows0 if k % 2 == 0 else rows1
        sem = semg0 if k % 2 == 0 else semg1
        if k >= 2:
            descs[k - 2].wait()
        pltpu.sync_copy(agg_sh.at[pl.ds(rr, ZCH), :], buf.at[pl.ds(0, ZCH), :])
        descs[k] = pltpu.async_copy(buf.at[pl.ds(0, ZCH), :],
                                    out_hbm.at[c, pl.ds(rr, ZCH), :], sem)
    descs[nd - 2].wait()
    descs[nd - 1].wait()


@functools.partial(
    pl.kernel,
    out_type=jax.ShapeDtypeStruct((NC, AGGN, H), jnp.float32),
    mesh=plsc.VectorSubcoreMesh(core_axis_name="c", subcore_axis_name="s"),
    scratch_types=[
        pltpu.VMEM_SHARED((AGGN, H), jnp.float32),
        pltpu.VMEM((NCH, CH), jnp.int32),
        pltpu.VMEM((2, CH), jnp.int32),
        pltpu.VMEM((2, CH), jnp.int32),
        pltpu.VMEM((CH, H), jnp.float32),
        pltpu.VMEM((CH, H), jnp.float32),
        pltpu.SemaphoreType.DMA,
        pltpu.SemaphoreType.DMA,
    ],
)
def _sc_segsum_call(h_hbm, packed_hbm, out_hbm,
                    agg_sh, pidx, u0, u1, rows0, rows1, semg0, semg1):
    _sc_body(h_hbm, packed_hbm, out_hbm,
             agg_sh, pidx, u0, u1, rows0, rows1, semg0, semg1)


def _segsum(h, packed3):
    return _sc_segsum_call(h, packed3)


def kernel(x, edge_index, batch, pre_W, pre_b, convW1, convb1, convW2,
           convb2, bn_g, bn_b, postW1, postb1, postW2, postb2):
    num_genes, emb = 1000, 64
    L = convW1.shape[0]
    src3 = edge_index[0].reshape(NW, NCH, CH)
    dst3 = edge_index[1].reshape(NW, NCH, CH)
    packed3 = jnp.bitwise_or(src3, jnp.left_shift(dst3, 14))

    h = _tc_pre(x, pre_W.T, pre_b.reshape(1, -1))
    for i in range(L - 1):
        parts = _segsum(h, packed3)
        h_raw, st = _tc_layer(h, parts, convW1[i].T,
                              convb1[i].reshape(1, -1), convW2[i].T,
                              convb2[i].reshape(1, -1))
        h = _tc_bn(h_raw, st, bn_g[i].reshape(1, -1),
                   bn_b[i].reshape(1, -1))
    parts = _segsum(h, packed3)
    out = _tc_layer_post(h, parts, convW1[L - 1].T,
                         convb1[L - 1].reshape(1, -1), convW2[L - 1].T,
                         convb2[L - 1].reshape(1, -1),
                         postW1.T, postb1.reshape(1, -1), postW2.T,
                         postb2.reshape(1, -1))
    return out.reshape(-1, num_genes * emb)
